# baseline (speedup 1.0000x reference)
import jax
import jax.numpy as jnp
from jax.experimental import pallas as pl
from jax.experimental.pallas import tpu as pltpu

_IMGS = 2
_KT = 6
_NJ = 4


def _shift_rows(a, s):
    if s == 0:
        return a
    z = jnp.zeros((1, a.shape[1]), a.dtype)
    if s == 1:
        return jnp.concatenate([a[1:], z], axis=0)
    return jnp.concatenate([z, a[:-1]], axis=0)


def _conv3x3_pipelined_kernel(x_ref, m_ref, b_ref, o_ref, lhs_ref):
    imgs, cin, H, W = x_ref.shape
    cout = o_ref.shape[1]
    n = pl.program_id(0)
    last = pl.num_programs(0) - 1

    @pl.when(n == 0)
    def _drain():
        for mxu in range(2):
            for base in range(0, 256, 64):
                pltpu.matmul_pop(base, (256, 256), jnp.float32, mxu)

    def _matmuls(bank, lhs_buf):
        for kk in range(_KT):
            tile = lhs_ref[lhs_buf, :, kk * 256:(kk + 1) * 256]
            for c in range(_NJ // 2):
                for mxu in range(2):
                    nj = c * 2 + mxu
                    reg = (kk * 2 + c) % 2
                    pltpu.matmul_push_rhs(
                        m_ref[kk * 256:(kk + 1) * 256,
                              nj * 256:(nj + 1) * 256],
                        staging_register=reg, mxu_index=mxu)
                    pltpu.matmul_acc_lhs(bank + c * 64, tile, mxu_index=mxu,
                                         load_staged_rhs=reg)

    @pl.when(jnp.logical_and(jnp.logical_and(n >= 1, n <= last - 1),
                             (n - 1) % 2 == 0))
    def _mm_even():
        _matmuls(0, 0)

    @pl.when(jnp.logical_and(jnp.logical_and(n >= 1, n <= last - 1),
                             (n - 1) % 2 == 1))
    def _mm_odd():
        _matmuls(128, 1)

    @pl.when(n <= last - 2)
    def _prologue():
        p = n % 2
        for i in range(imgs):
            for ci in range(cin):
                base = x_ref[i, ci]
                for dh in range(3):
                    k = ci * 3 + dh
                    lhs_ref[p, i * H:(i + 1) * H, k * W:(k + 1) * W] = (
                        _shift_rows(base, dh - 1))

    def _epilogue(bank):
        for c in range(_NJ // 2):
            for mxu in range(2):
                nj = c * 2 + mxu
                vals = pltpu.matmul_pop(bank + c * 64, (256, 256),
                                        jnp.float32, mxu)
                for i in range(imgs):
                    for s in range(2):
                        co = nj * 2 + s
                        o_ref[i, co] = (
                            vals[i * H:(i + 1) * H, s * W:(s + 1) * W]
                            + b_ref[co])

    @pl.when(jnp.logical_and(n >= 2, n % 2 == 0))
    def _ep_even():
        _epilogue(0)

    @pl.when(jnp.logical_and(n >= 2, n % 2 == 1))
    def _ep_odd():
        _epilogue(128)


def _build_w_toeplitz(weight_oihw, W):
    p = jnp.arange(W)
    dw = jnp.arange(3)
    sel = (p[None, :, None] ==
           (p[None, None, :] + dw[:, None, None] - 1)).astype(jnp.float32)
    cout, cin = weight_oihw.shape[:2]
    m = jnp.einsum("dpq,ockd->ckpoq", sel, weight_oihw)
    return m.reshape(cin * 3 * W, cout * W)


@jax.jit
def _conv_impl(x_nchw, weight_oihw, bias):
    N, Cin, H, W = x_nchw.shape
    Cout = weight_oihw.shape[0]
    npairs = N // _IMGS
    m = _build_w_toeplitz(weight_oihw, W)
    return pl.pallas_call(
        _conv3x3_pipelined_kernel,
        out_shape=jax.ShapeDtypeStruct((N, Cout, H, W), jnp.float32),
        grid=(npairs + 2,),
        in_specs=[
            pl.BlockSpec((_IMGS, Cin, H, W),
                         lambda n: (jnp.minimum(n, npairs - 1), 0, 0, 0)),
            pl.BlockSpec((Cin * 3 * W, Cout * W), lambda n: (0, 0)),
            pl.BlockSpec(memory_space=pltpu.SMEM),
        ],
        out_specs=pl.BlockSpec((_IMGS, Cout, H, W),
                               lambda n: (jnp.maximum(n - 2, 0), 0, 0, 0)),
        scratch_shapes=[pltpu.VMEM((2, _IMGS * H, Cin * 3 * W), jnp.float32)],
        compiler_params=pltpu.CompilerParams(
            dimension_semantics=("arbitrary",),
            vmem_limit_bytes=48 * 1024 * 1024,
        ),
    )(x_nchw, m, bias).astype(x_nchw.dtype)


def kernel(x_nchw, weight_oihw, bias):
    return _conv_impl(x_nchw, weight_oihw, bias)


# speedup vs baseline: 1.0944x; 1.0944x over previous
import jax
import jax.numpy as jnp
from jax.experimental import pallas as pl
from jax.experimental.pallas import tpu as pltpu

_IMGS = 2
_KT = 6
_NJ = 4


def _shift_rows(a, s):
    if s == 0:
        return a
    z = jnp.zeros((1, a.shape[1]), a.dtype)
    if s == 1:
        return jnp.concatenate([a[1:], z], axis=0)
    return jnp.concatenate([z, a[:-1]], axis=0)


def _conv3x3_pipelined_kernel(x1_ref, x2_ref, m_ref, b_ref, o_ref,
                              lhsa_ref, lhsb_ref):
    _, cin, H, W = x1_ref.shape
    cout = o_ref.shape[1]
    k = pl.program_id(0)
    last = pl.num_programs(0) - 1

    def _prologue(lhs_ref, x_ref, img_off):
        for i in range(_IMGS):
            for ci in range(cin):
                base = x_ref[img_off + i, ci]
                for dh in range(3):
                    kk = ci * 3 + dh
                    lhs_ref[i * H:(i + 1) * H, kk * W:(kk + 1) * W] = (
                        _shift_rows(base, dh - 1))

    def _matmuls(bank, lhs_ref):
        for kk in range(_KT):
            tile = lhs_ref[:, kk * 256:(kk + 1) * 256]
            for c in range(_NJ // 2):
                for mxu in range(2):
                    nj = c * 2 + mxu
                    reg = (kk + c) % 2
                    pltpu.matmul_push_rhs(
                        m_ref[kk * 256:(kk + 1) * 256,
                              nj * 256:(nj + 1) * 256],
                        staging_register=reg, mxu_index=mxu)
                    pltpu.matmul_acc_lhs(bank + c * 64, tile, mxu_index=mxu,
                                         load_staged_rhs=reg)

    def _epilogue(bank, img_off):
        for c in range(_NJ // 2):
            for mxu in range(2):
                nj = c * 2 + mxu
                vals = pltpu.matmul_pop(bank + c * 64, (256, 256),
                                        jnp.float32, mxu)
                for i in range(_IMGS):
                    for s in range(2):
                        co = nj * 2 + s
                        o_ref[img_off + i, co] = (
                            vals[i * H:(i + 1) * H, s * W:(s + 1) * W]
                            + b_ref[co])

    @pl.when(k == 0)
    def _warmup():
        _prologue(lhsa_ref, x1_ref, 0)

    _epilogue(0, 0)
    _matmuls(0, lhsa_ref)
    _prologue(lhsb_ref, x1_ref, 2)
    _epilogue(128, 2)
    _matmuls(128, lhsb_ref)
    _prologue(lhsa_ref, x2_ref, 0)

    @pl.when(k == last)
    def _drain():
        for mxu in range(2):
            for base in range(0, 256, 64):
                pltpu.matmul_pop(base, (256, 256), jnp.float32, mxu)


def _build_w_toeplitz(weight_oihw, W):
    p = jnp.arange(W)
    dw = jnp.arange(3)
    sel = (p[None, :, None] ==
           (p[None, None, :] + dw[:, None, None] - 1)).astype(jnp.float32)
    cout, cin = weight_oihw.shape[:2]
    m = jnp.einsum("dpq,ockd->ckpoq", sel, weight_oihw)
    return m.reshape(cin * 3 * W, cout * W)


@jax.jit
def _conv_impl(x_nchw, weight_oihw, bias):
    N, Cin, H, W = x_nchw.shape
    Cout = weight_oihw.shape[0]
    nblocks = N // 4
    m = _build_w_toeplitz(weight_oihw, W)
    grid = (nblocks + 1,)
    return pl.pallas_call(
        _conv3x3_pipelined_kernel,
        out_shape=jax.ShapeDtypeStruct((N, Cout, H, W), jnp.float32),
        grid=grid,
        in_specs=[
            pl.BlockSpec((4, Cin, H, W),
                         lambda k: (jnp.minimum(k, nblocks - 1), 0, 0, 0)),
            pl.BlockSpec((4, Cin, H, W),
                         lambda k: (jnp.minimum(k + 1, nblocks - 1), 0, 0, 0)),
            pl.BlockSpec((Cin * 3 * W, Cout * W), lambda k: (0, 0)),
            pl.BlockSpec(memory_space=pltpu.SMEM),
        ],
        out_specs=pl.BlockSpec((4, Cout, H, W),
                               lambda k: (jnp.maximum(k - 1, 0), 0, 0, 0)),
        scratch_shapes=[
            pltpu.VMEM((_IMGS * H, Cin * 3 * W), jnp.float32),
            pltpu.VMEM((_IMGS * H, Cin * 3 * W), jnp.float32),
        ],
        compiler_params=pltpu.CompilerParams(
            dimension_semantics=("arbitrary",),
            vmem_limit_bytes=48 * 1024 * 1024,
        ),
    )(x_nchw, x_nchw, m, bias).astype(x_nchw.dtype)


def kernel(x_nchw, weight_oihw, bias):
    return _conv_impl(x_nchw, weight_oihw, bias)


# speedup vs baseline: 2.0214x; 1.8471x over previous
import jax
import jax.numpy as jnp
from jax.experimental import pallas as pl
from jax.experimental.pallas import tpu as pltpu

_IMGS = 2
_KT = 6
_NJ = 4


def _shift_rows(a, s):
    if s == 0:
        return a
    z = jnp.zeros((1, a.shape[1]), a.dtype)
    if s == 1:
        return jnp.concatenate([a[1:], z], axis=0)
    return jnp.concatenate([z, a[:-1]], axis=0)


def _conv3x3_pipelined_kernel(x1_ref, x2_ref, m_ref, b_ref, o_ref,
                              lhsa_ref, lhsb_ref):
    _, cin, H, W = x1_ref.shape
    cout = o_ref.shape[1]
    k = pl.program_id(0)
    last = pl.num_programs(0) - 1

    def _prologue(lhs_ref, x_ref, img_off):
        for i in range(_IMGS):
            for ci in range(cin):
                base = x_ref[img_off + i, ci]
                for dh in range(3):
                    kk = ci * 3 + dh
                    lhs_ref[i * H:(i + 1) * H, kk * W:(kk + 1) * W] = (
                        _shift_rows(base, dh - 1).astype(jnp.bfloat16))

    def _matmuls(bank, lhs_ref):
        for kk in range(_KT):
            tile = lhs_ref[:, kk * 256:(kk + 1) * 256]
            for c in range(_NJ // 2):
                for mxu in range(2):
                    nj = c * 2 + mxu
                    reg = (kk + c) % 2
                    pltpu.matmul_push_rhs(
                        m_ref[kk * 256:(kk + 1) * 256,
                              nj * 256:(nj + 1) * 256],
                        staging_register=reg, mxu_index=mxu)
                    pltpu.matmul_acc_lhs(bank + c * 64, tile, mxu_index=mxu,
                                         load_staged_rhs=reg)

    def _epilogue(bank, img_off):
        for c in range(_NJ // 2):
            for mxu in range(2):
                nj = c * 2 + mxu
                vals = pltpu.matmul_pop(bank + c * 64, (256, 256),
                                        jnp.float32, mxu)
                for i in range(_IMGS):
                    for s in range(2):
                        co = nj * 2 + s
                        o_ref[img_off + i, co] = (
                            vals[i * H:(i + 1) * H, s * W:(s + 1) * W]
                            + b_ref[co])

    @pl.when(k == 0)
    def _warmup():
        _prologue(lhsa_ref, x1_ref, 0)

    _epilogue(0, 0)
    _matmuls(0, lhsa_ref)
    _prologue(lhsb_ref, x1_ref, 2)
    _epilogue(128, 2)
    _matmuls(128, lhsb_ref)
    _prologue(lhsa_ref, x2_ref, 0)

    @pl.when(k == last)
    def _drain():
        for mxu in range(2):
            for base in range(0, 256, 64):
                pltpu.matmul_pop(base, (256, 256), jnp.float32, mxu)


def _build_w_toeplitz(weight_oihw, W):
    p = jnp.arange(W)
    dw = jnp.arange(3)
    sel = (p[None, :, None] ==
           (p[None, None, :] + dw[:, None, None] - 1)).astype(jnp.float32)
    cout, cin = weight_oihw.shape[:2]
    m = jnp.einsum("dpq,ockd->ckpoq", sel, weight_oihw)
    return m.reshape(cin * 3 * W, cout * W).astype(jnp.bfloat16)


@jax.jit
def _conv_impl(x_nchw, weight_oihw, bias):
    N, Cin, H, W = x_nchw.shape
    Cout = weight_oihw.shape[0]
    nblocks = N // 4
    m = _build_w_toeplitz(weight_oihw, W)
    grid = (nblocks + 1,)
    return pl.pallas_call(
        _conv3x3_pipelined_kernel,
        out_shape=jax.ShapeDtypeStruct((N, Cout, H, W), jnp.float32),
        grid=grid,
        in_specs=[
            pl.BlockSpec((4, Cin, H, W),
                         lambda k: (jnp.minimum(k, nblocks - 1), 0, 0, 0)),
            pl.BlockSpec((4, Cin, H, W),
                         lambda k: (jnp.minimum(k + 1, nblocks - 1), 0, 0, 0)),
            pl.BlockSpec((Cin * 3 * W, Cout * W), lambda k: (0, 0)),
            pl.BlockSpec(memory_space=pltpu.SMEM),
        ],
        out_specs=pl.BlockSpec((4, Cout, H, W),
                               lambda k: (jnp.maximum(k - 1, 0), 0, 0, 0)),
        scratch_shapes=[
            pltpu.VMEM((_IMGS * H, Cin * 3 * W), jnp.bfloat16),
            pltpu.VMEM((_IMGS * H, Cin * 3 * W), jnp.bfloat16),
        ],
        compiler_params=pltpu.CompilerParams(
            dimension_semantics=("arbitrary",),
            vmem_limit_bytes=48 * 1024 * 1024,
        ),
    )(x_nchw, x_nchw, m, bias).astype(x_nchw.dtype)


def kernel(x_nchw, weight_oihw, bias):
    return _conv_impl(x_nchw, weight_oihw, bias)


# speedup vs baseline: 2.0267x; 1.0026x over previous
import jax
import jax.numpy as jnp
from jax.experimental import pallas as pl
from jax.experimental.pallas import tpu as pltpu

_IMGS = 2
_KT = 6
_NJ = 4


def _shift_rows(a, s):
    if s == 0:
        return a
    z = jnp.zeros((1, a.shape[1]), a.dtype)
    if s == 1:
        return jnp.concatenate([a[1:], z], axis=0)
    return jnp.concatenate([z, a[:-1]], axis=0)


def _conv3x3_pipelined_kernel(x1_ref, x2_ref, m_ref, b_ref, o_ref,
                              lhsa_ref, lhsb_ref):
    _, cin, H, W = x1_ref.shape
    cout = o_ref.shape[1]
    k = pl.program_id(0)
    last = pl.num_programs(0) - 1

    def _prologue(lhs_ref, x_ref, img_off):
        for i in range(_IMGS):
            for ci in range(cin):
                base = x_ref[img_off + i, ci]
                for dh in range(3):
                    kk = ci * 3 + dh
                    lhs_ref[i * H:(i + 1) * H, kk * W:(kk + 1) * W] = (
                        _shift_rows(base, dh - 1).astype(jnp.bfloat16))

    def _matmuls():
        for kk in range(_KT):
            tile_a = lhsa_ref[:, kk * 256:(kk + 1) * 256]
            tile_b = lhsb_ref[:, kk * 256:(kk + 1) * 256]
            for c in range(_NJ // 2):
                for mxu in range(2):
                    nj = c * 2 + mxu
                    reg = (kk + c) % 2
                    pltpu.matmul_push_rhs(
                        m_ref[kk * 256:(kk + 1) * 256,
                              nj * 256:(nj + 1) * 256],
                        staging_register=reg, mxu_index=mxu)
                    pltpu.matmul_acc_lhs(c * 64, tile_a, mxu_index=mxu,
                                         load_staged_rhs=reg)
                    pltpu.matmul_acc_lhs(128 + c * 64, tile_b, mxu_index=mxu,
                                         load_staged_rhs=None)

    def _epilogue(bank, img_off):
        for c in range(_NJ // 2):
            for mxu in range(2):
                nj = c * 2 + mxu
                vals = pltpu.matmul_pop(bank + c * 64, (256, 256),
                                        jnp.float32, mxu)
                for i in range(_IMGS):
                    for s in range(2):
                        co = nj * 2 + s
                        o_ref[img_off + i, co] = (
                            vals[i * H:(i + 1) * H, s * W:(s + 1) * W]
                            + b_ref[co])

    @pl.when(k == 0)
    def _warmup():
        _prologue(lhsa_ref, x1_ref, 0)
        _prologue(lhsb_ref, x1_ref, 2)

    _epilogue(0, 0)
    _epilogue(128, 2)
    _matmuls()
    _prologue(lhsa_ref, x2_ref, 0)
    _prologue(lhsb_ref, x2_ref, 2)

    @pl.when(k == last)
    def _drain():
        for mxu in range(2):
            for base in range(0, 256, 64):
                pltpu.matmul_pop(base, (256, 256), jnp.float32, mxu)


def _build_w_toeplitz(weight_oihw, W):
    p = jnp.arange(W)
    dw = jnp.arange(3)
    sel = (p[None, :, None] ==
           (p[None, None, :] + dw[:, None, None] - 1)).astype(jnp.float32)
    cout, cin = weight_oihw.shape[:2]
    m = jnp.einsum("dpq,ockd->ckpoq", sel, weight_oihw)
    return m.reshape(cin * 3 * W, cout * W).astype(jnp.bfloat16)


@jax.jit
def _conv_impl(x_nchw, weight_oihw, bias):
    N, Cin, H, W = x_nchw.shape
    Cout = weight_oihw.shape[0]
    nblocks = N // 4
    m = _build_w_toeplitz(weight_oihw, W)
    grid = (nblocks + 1,)
    return pl.pallas_call(
        _conv3x3_pipelined_kernel,
        out_shape=jax.ShapeDtypeStruct((N, Cout, H, W), jnp.float32),
        grid=grid,
        in_specs=[
            pl.BlockSpec((4, Cin, H, W),
                         lambda k: (jnp.minimum(k, nblocks - 1), 0, 0, 0)),
            pl.BlockSpec((4, Cin, H, W),
                         lambda k: (jnp.minimum(k + 1, nblocks - 1), 0, 0, 0)),
            pl.BlockSpec((Cin * 3 * W, Cout * W), lambda k: (0, 0)),
            pl.BlockSpec(memory_space=pltpu.SMEM),
        ],
        out_specs=pl.BlockSpec((4, Cout, H, W),
                               lambda k: (jnp.maximum(k - 1, 0), 0, 0, 0)),
        scratch_shapes=[
            pltpu.VMEM((_IMGS * H, Cin * 3 * W), jnp.bfloat16),
            pltpu.VMEM((_IMGS * H, Cin * 3 * W), jnp.bfloat16),
        ],
        compiler_params=pltpu.CompilerParams(
            dimension_semantics=("arbitrary",),
            vmem_limit_bytes=48 * 1024 * 1024,
        ),
    )(x_nchw, x_nchw, m, bias).astype(x_nchw.dtype)


def kernel(x_nchw, weight_oihw, bias):
    return _conv_impl(x_nchw, weight_oihw, bias)


# speedup vs baseline: 2.1728x; 1.0721x over previous
import jax
import jax.numpy as jnp
from jax.experimental import pallas as pl
from jax.experimental.pallas import tpu as pltpu

_IMGS = 2
_KT = 6
_NJ = 4


def _shift_rows(a, s):
    if s == 0:
        return a
    z = jnp.zeros((1, a.shape[1]), a.dtype)
    if s == 1:
        return jnp.concatenate([a[1:], z], axis=0)
    return jnp.concatenate([z, a[:-1]], axis=0)


def _conv3x3_pipelined_kernel(x1_ref, x2_ref, m_ref, b_ref, o_ref,
                              lhsa_ref, lhsb_ref):
    _, cin, H, W = x1_ref.shape
    cout = o_ref.shape[1]
    k = pl.program_id(0)
    last = pl.num_programs(0) - 1

    def _prologue(lhs_ref, x_ref, img_off):
        for i in range(_IMGS):
            for ci in range(cin):
                base = x_ref[img_off + i, ci]
                for dh in range(3):
                    kk = ci * 3 + dh
                    lhs_ref[i * H:(i + 1) * H, kk * W:(kk + 1) * W] = (
                        _shift_rows(base, dh - 1).astype(jnp.bfloat16))

    def _matmuls():
        for kk in range(_KT):
            tile_a = lhsa_ref[:, kk * 256:(kk + 1) * 256]
            tile_b = lhsb_ref[:, kk * 256:(kk + 1) * 256]
            for c in range(_NJ // 2):
                for mxu in range(2):
                    nj = c * 2 + mxu
                    reg = (kk + c) % 2
                    pltpu.matmul_push_rhs(
                        m_ref[kk * 256:(kk + 1) * 256,
                              nj * 256:(nj + 1) * 256],
                        staging_register=reg, mxu_index=mxu)
                    pltpu.matmul_acc_lhs(c * 64, tile_a, mxu_index=mxu,
                                         load_staged_rhs=reg)
                    pltpu.matmul_acc_lhs(128 + c * 64, tile_b, mxu_index=mxu,
                                         load_staged_rhs=None)

    def _epilogue(bank, img_off):
        for c in range(_NJ // 2):
            for mxu in range(2):
                nj = c * 2 + mxu
                vals = pltpu.matmul_pop(bank + c * 64, (256, 256),
                                        jnp.float32, mxu)
                for i in range(_IMGS):
                    for s in range(2):
                        co = nj * 2 + s
                        o_ref[img_off + i, co] = (
                            vals[i * H:(i + 1) * H, s * W:(s + 1) * W]
                            + b_ref[co])

    @pl.when(k == 0)
    def _warmup():
        _prologue(lhsa_ref, x1_ref, 0)
        _prologue(lhsb_ref, x1_ref, 2)

    _epilogue(0, 0)
    _epilogue(128, 2)
    _matmuls()
    _prologue(lhsa_ref, x2_ref, 0)
    _prologue(lhsb_ref, x2_ref, 2)

    @pl.when(k == last)
    def _drain():
        for mxu in range(2):
            for base in range(0, 256, 64):
                pltpu.matmul_pop(base, (256, 256), jnp.float32, mxu)


def _build_w_toeplitz(weight_oihw, W):
    p = jnp.arange(W)
    dw = jnp.arange(3)
    sel = (p[None, :, None] ==
           (p[None, None, :] + dw[:, None, None] - 1)).astype(jnp.bfloat16)
    cout, cin = weight_oihw.shape[:2]
    m = jnp.einsum("dpq,ockd->ckpoq", sel,
                   weight_oihw.astype(jnp.bfloat16),
                   preferred_element_type=jnp.bfloat16)
    return m.reshape(cin * 3 * W, cout * W)


@jax.jit
def _conv_impl(x_nchw, weight_oihw, bias):
    N, Cin, H, W = x_nchw.shape
    Cout = weight_oihw.shape[0]
    nblocks = N // 4
    m = _build_w_toeplitz(weight_oihw, W)
    grid = (nblocks + 1,)
    return pl.pallas_call(
        _conv3x3_pipelined_kernel,
        out_shape=jax.ShapeDtypeStruct((N, Cout, H, W), jnp.float32),
        grid=grid,
        in_specs=[
            pl.BlockSpec((4, Cin, H, W), lambda k: (0, 0, 0, 0)),
            pl.BlockSpec((4, Cin, H, W),
                         lambda k: (jnp.minimum(k + 1, nblocks - 1), 0, 0, 0)),
            pl.BlockSpec((Cin * 3 * W, Cout * W), lambda k: (0, 0)),
            pl.BlockSpec(memory_space=pltpu.SMEM),
        ],
        out_specs=pl.BlockSpec((4, Cout, H, W),
                               lambda k: (jnp.maximum(k - 1, 0), 0, 0, 0)),
        scratch_shapes=[
            pltpu.VMEM((_IMGS * H, Cin * 3 * W), jnp.bfloat16),
            pltpu.VMEM((_IMGS * H, Cin * 3 * W), jnp.bfloat16),
        ],
        compiler_params=pltpu.CompilerParams(
            dimension_semantics=("arbitrary",),
            vmem_limit_bytes=48 * 1024 * 1024,
        ),
    )(x_nchw, x_nchw, m, bias).astype(x_nchw.dtype)


def kernel(x_nchw, weight_oihw, bias):
    return _conv_impl(x_nchw, weight_oihw, bias)
